# Initial kernel scaffold; baseline (speedup 1.0000x reference)
import jax
import jax.numpy as jnp
from jax.experimental import pallas as pl


def kernel(x, edge_index, W, b):
    raise NotImplementedError("write your pallas kernel here")


# speedup vs baseline: 6.2604x; 6.2604x over previous
import functools

import jax
import jax.numpy as jnp
from jax import lax
from jax.experimental import pallas as pl
from jax.experimental.pallas import tpu as pltpu
from jax.experimental.pallas import tpu_sc as plsc

N_NODES = 10000
D = 128
E = 320000

NC, NS = 2, 16
NW = NC * NS
N_PAD = 10240
ROWS_PER_SUB = N_PAD // NS
UNIT = 128
UNITS = -(-E // (NW * UNIT))
EPW_PAD = UNITS * UNIT


def _sc_aggregate():
    mesh = plsc.VectorSubcoreMesh(core_axis_name="c", subcore_axis_name="s")

    @functools.partial(
        pl.kernel,
        mesh=mesh,
        out_type=[
            jax.ShapeDtypeStruct((NC, N_PAD, D), jnp.float32),
            jax.ShapeDtypeStruct((NC, N_PAD), jnp.float32),
        ],
        scratch_types=[
            pltpu.VMEM_SHARED((N_PAD, D), jnp.float32),
            pltpu.VMEM_SHARED((N_PAD,), jnp.float32),
            pltpu.VMEM((UNITS, UNIT), jnp.int32),
            pltpu.VMEM((UNITS, UNIT), jnp.int32),
            pltpu.VMEM((UNIT, D), jnp.float32),
            pltpu.VMEM((UNIT,), jnp.float32),
            pltpu.VMEM((ROWS_PER_SUB,), jnp.float32),
            pltpu.SemaphoreType.DMA,
        ],
    )
    def sc_agg(x_h, src_h, dst_h, sums_h, deg_h,
               sums_sh, deg_sh, src_v, dst_v, rows_v, ones_v, zcol_v, sem):
        c = lax.axis_index("c")
        s = lax.axis_index("s")
        w = c * NS + s

        for i in range(UNIT // 16):
            ones_v[pl.ds(i * 16, 16)] = jnp.full((16,), 1.0, jnp.float32)

        def zrow(i, carry):
            for k in range(D // 16):
                rows_v[i, pl.ds(k * 16, 16)] = jnp.zeros((16,), jnp.float32)
            return carry
        lax.fori_loop(0, UNIT, zrow, 0)

        def zcol(i, carry):
            zcol_v[pl.ds(i * 16, 16)] = jnp.zeros((16,), jnp.float32)
            return carry
        lax.fori_loop(0, ROWS_PER_SUB // 16, zcol, 0)

        for k in range(ROWS_PER_SUB // UNIT):
            pltpu.sync_copy(rows_v,
                            sums_sh.at[pl.ds(s * ROWS_PER_SUB + k * UNIT, UNIT)])
        pltpu.sync_copy(zcol_v, deg_sh.at[pl.ds(s * ROWS_PER_SUB, ROWS_PER_SUB)])
        plsc.subcore_barrier()

        pltpu.sync_copy(src_h.at[w], src_v)
        pltpu.sync_copy(dst_h.at[w], dst_v)

        def body(j, carry):
            pltpu.async_copy(x_h.at[src_v.at[j]], rows_v, sem).wait()
            pltpu.sync_copy(rows_v, sums_sh.at[dst_v.at[j]], add=True)
            pltpu.sync_copy(ones_v, deg_sh.at[dst_v.at[j]], add=True)
            return carry
        lax.fori_loop(0, UNITS, body, 0)
        plsc.subcore_barrier()

        pltpu.sync_copy(sums_sh.at[pl.ds(s * ROWS_PER_SUB, ROWS_PER_SUB)],
                        sums_h.at[c, pl.ds(s * ROWS_PER_SUB, ROWS_PER_SUB)])
        pltpu.sync_copy(deg_sh.at[pl.ds(s * ROWS_PER_SUB, ROWS_PER_SUB)],
                        deg_h.at[c, pl.ds(s * ROWS_PER_SUB, ROWS_PER_SUB)])

    return sc_agg


_SC_AGG = _sc_aggregate()

BLK = 1024
GRID = N_PAD // BLK


def _tc_body(x_ref, sums_ref, degb_ref, w_ref, b_ref, o_ref):
    ssum = sums_ref[0] + sums_ref[1]
    deg = degb_ref[0] + degb_ref[1]
    agg = ssum * (1.0 / jnp.maximum(deg, 1.0))
    o_ref[...] = (
        jnp.dot(x_ref[...], w_ref[:D], preferred_element_type=jnp.float32)
        + jnp.dot(agg, w_ref[D:], preferred_element_type=jnp.float32)
        + b_ref[...]
    )


_TC_FC = pl.pallas_call(
    _tc_body,
    grid=(GRID,),
    in_specs=[
        pl.BlockSpec((BLK, D), lambda i: (i, 0)),
        pl.BlockSpec((NC, BLK, D), lambda i: (0, i, 0)),
        pl.BlockSpec((NC, BLK, D), lambda i: (0, i, 0)),
        pl.BlockSpec((2 * D, D), lambda i: (0, 0)),
        pl.BlockSpec((1, D), lambda i: (0, 0)),
    ],
    out_specs=pl.BlockSpec((BLK, D), lambda i: (i, 0)),
    out_shape=jax.ShapeDtypeStruct((N_PAD, D), jnp.float32),
)


def kernel(x, edge_index, W, b):
    src = edge_index[0].astype(jnp.int32)
    dst = edge_index[1].astype(jnp.int32)
    pad = NW * EPW_PAD - E
    src_p = jnp.concatenate([src, jnp.zeros((pad,), jnp.int32)]).reshape(
        NW, UNITS, UNIT)
    dst_p = jnp.concatenate([dst, jnp.full((pad,), N_PAD - 1, jnp.int32)]
                            ).reshape(NW, UNITS, UNIT)
    sums, deg = _SC_AGG(x, src_p, dst_p)
    x_p = jnp.pad(x, ((0, N_PAD - N_NODES), (0, 0)))
    deg_b = jnp.broadcast_to(deg[:, :, None], (NC, N_PAD, D))
    out = _TC_FC(x_p, sums, deg_b, W, b.reshape(1, D))
    return out[:N_NODES]
